# baseline (speedup 1.0000x reference)
import functools

import jax
import jax.numpy as jnp
from jax import lax
from jax.experimental import pallas as pl
from jax.experimental.pallas import tpu as pltpu
from jax.experimental.pallas import tpu_sc as plsc


def _sc_dots_kernel(nc, ns, V, D, C, spt, table_t, vi, vj, ng):
    dpc = D // nc
    mesh = plsc.VectorSubcoreMesh(core_axis_name="c", subcore_axis_name="s")

    @functools.partial(
        pl.kernel,
        mesh=mesh,
        compiler_params=pltpu.CompilerParams(
            use_tc_tiling_on_sc=True, needs_layout_passes=False
        ),
        out_type=jax.ShapeDtypeStruct((nc, ns, C, spt), jnp.float32),
        scratch_types=[
            pltpu.VMEM_SHARED((V,), jnp.float32),
            pltpu.VMEM((C + 1, 1, spt), jnp.int32),
            pltpu.VMEM((C + 1, 1, spt), jnp.float32),
            pltpu.VMEM((C, 1, spt), jnp.float32),
            pltpu.SemaphoreType.DMA,
            pltpu.SemaphoreType.DMA,
            pltpu.SemaphoreType.DMA,
        ],
    )
    def k(tab, vi_h, vj_h, ng_h, out_h, sp0, idx_v, val_v, acc_v,
          semA, semB, semG):
        cid = lax.axis_index("c")
        sid = lax.axis_index("s")
        d0 = cid * dpc
        s_lo = sid * spt
        pltpu.sync_copy(vi_h.at[pl.ds(s_lo, spt)], idx_v.at[0, 0])
        pltpu.sync_copy(vj_h.at[pl.ds(s_lo, spt)], idx_v.at[1, 0])
        pltpu.sync_copy(
            ng_h.at[:, :, pl.ds(s_lo, spt)], idx_v.at[pl.ds(2, C - 1)]
        )

        zeros16 = jnp.zeros((16,), jnp.float32)

        def zblk(b, _):
            for c in range(C):
                acc_v[c, 0, pl.ds(b * 16, 16)] = zeros16
            return 0

        lax.fori_loop(0, spt // 16, zblk, 0)

        def stage(d):
            pltpu.async_copy(tab.at[d], sp0, semA)

        def drain_row():
            pltpu.make_async_copy(tab.at[0], sp0, semA).wait()

        @pl.when(sid == 0)
        def _():
            stage(d0)

        def d_body(dl, _):
            @pl.when(sid == 0)
            def _():
                drain_row()

            plsc.subcore_barrier()

            cps = [
                pltpu.async_copy(sp0.at[idx_v.at[c, 0]], val_v.at[c, 0], semG)
                for c in range(C + 1)
            ]
            for cp in cps:
                cp.wait()

            plsc.subcore_barrier()

            @pl.when((sid == 0) & (dl + 1 < dpc))
            def _():
                stage(d0 + dl + 1)

            def blk(b, _):
                s0 = b * 16
                v0 = val_v[0, 0, pl.ds(s0, 16)]
                for c in range(C):
                    acc_v[c, 0, pl.ds(s0, 16)] = (
                        acc_v[c, 0, pl.ds(s0, 16)]
                        + v0 * val_v[c + 1, 0, pl.ds(s0, 16)]
                    )
                return 0

            lax.fori_loop(0, spt // 16, blk, 0)
            return 0

        lax.fori_loop(0, dpc, d_body, 0)

        for c in range(C):
            pltpu.sync_copy(acc_v.at[c, 0], out_h.at[cid, sid, c])

    return k(table_t, vi, vj, ng)


def _tc_loss_kernel(parts, batch):
    def body(x_ref, o_ref):
        x = x_ref[0] + x_ref[1]
        ls = jnp.minimum(x, 0.0) - jnp.log1p(jnp.exp(-jnp.abs(x)))
        o_ref[0, 0] = -jnp.sum(ls) / batch

    return pl.pallas_call(
        body,
        out_shape=jax.ShapeDtypeStruct((1, 1), jnp.float32),
        out_specs=pl.BlockSpec(memory_space=pltpu.SMEM),
    )(parts)


def kernel(v_i, v_j, negsamples, device, first_embeddings):
    B = v_i.shape[0]
    V, D = first_embeddings.shape
    C = negsamples.shape[0] + 1

    info = plsc.get_sparse_core_info()
    nc, ns = info.num_cores, info.num_subcores
    spt = B // ns

    parts = _sc_dots_kernel(
        nc, ns, V, D, C, spt, first_embeddings.T,
        v_i.astype(jnp.int32), v_j.astype(jnp.int32),
        negsamples.astype(jnp.int32).reshape(C - 1, 1, B),
    )
    out = _tc_loss_kernel(parts.reshape(nc, C * B // 1024, 1024), B)
    return out[0, 0]


# speedup vs baseline: 1.0031x; 1.0031x over previous
import functools

import jax
import jax.numpy as jnp
from jax import lax
from jax.experimental import pallas as pl
from jax.experimental.pallas import tpu as pltpu
from jax.experimental.pallas import tpu_sc as plsc


def _sc_dots_kernel(nc, ns, V, D, C, spt, table_t, idx_t):
    dpc = D // nc
    mesh = plsc.VectorSubcoreMesh(core_axis_name="c", subcore_axis_name="s")

    @functools.partial(
        pl.kernel,
        mesh=mesh,
        compiler_params=pltpu.CompilerParams(
            use_tc_tiling_on_sc=True, needs_layout_passes=False
        ),
        out_type=jax.ShapeDtypeStruct((nc, ns, C, spt), jnp.float32),
        scratch_types=[
            pltpu.VMEM_SHARED((V,), jnp.float32),
            pltpu.VMEM((C + 1, 1, spt), jnp.int32),
            pltpu.VMEM((C + 1, 1, spt), jnp.float32),
            pltpu.VMEM((C, 1, spt), jnp.float32),
            pltpu.SemaphoreType.DMA,
            pltpu.SemaphoreType.DMA,
            pltpu.SemaphoreType.DMA,
        ],
    )
    def k(tab, idx_h, out_h, sp0, idx_v, val_v, acc_v, semA, semB, semG):
        cid = lax.axis_index("c")
        sid = lax.axis_index("s")
        d0 = cid * dpc
        pltpu.sync_copy(idx_h.at[sid], idx_v)

        zeros16 = jnp.zeros((16,), jnp.float32)

        def zblk(b, _):
            for c in range(C):
                acc_v[c, 0, pl.ds(b * 16, 16)] = zeros16
            return 0

        lax.fori_loop(0, spt // 16, zblk, 0)

        def stage(d):
            pltpu.async_copy(tab.at[d], sp0, semA)

        def drain_row():
            pltpu.make_async_copy(tab.at[0], sp0, semA).wait()

        @pl.when(sid == 0)
        def _():
            stage(d0)

        def d_body(dl, _):
            @pl.when(sid == 0)
            def _():
                drain_row()

            plsc.subcore_barrier()

            cps = [
                pltpu.async_copy(sp0.at[idx_v.at[c, 0]], val_v.at[c, 0], semG)
                for c in range(C + 1)
            ]
            for cp in cps:
                cp.wait()

            plsc.subcore_barrier()

            @pl.when((sid == 0) & (dl + 1 < dpc))
            def _():
                stage(d0 + dl + 1)

            def blk(b, _):
                s0 = b * 16
                v0 = val_v[0, 0, pl.ds(s0, 16)]
                for c in range(C):
                    acc_v[c, 0, pl.ds(s0, 16)] = (
                        acc_v[c, 0, pl.ds(s0, 16)]
                        + v0 * val_v[c + 1, 0, pl.ds(s0, 16)]
                    )
                return 0

            lax.fori_loop(0, spt // 16, blk, 0)
            return 0

        lax.fori_loop(0, dpc, d_body, 0)

        for c in range(C):
            pltpu.sync_copy(acc_v.at[c, 0], out_h.at[cid, sid, c])

    return k(table_t, idx_t)


def _tc_loss_kernel(parts, batch):
    def body(x_ref, o_ref):
        x = x_ref[0] + x_ref[1]
        ls = jnp.minimum(x, 0.0) - jnp.log1p(jnp.exp(-jnp.abs(x)))
        o_ref[0, 0] = -jnp.sum(ls) / batch

    return pl.pallas_call(
        body,
        out_shape=jax.ShapeDtypeStruct((1, 1), jnp.float32),
        out_specs=pl.BlockSpec(memory_space=pltpu.SMEM),
    )(parts)


def kernel(v_i, v_j, negsamples, device, first_embeddings):
    B = v_i.shape[0]
    V, D = first_embeddings.shape
    C = negsamples.shape[0] + 1

    info = plsc.get_sparse_core_info()
    nc, ns = info.num_cores, info.num_subcores
    spt = B // ns

    all_idx = jnp.concatenate(
        [v_i[None].astype(jnp.int32), v_j[None].astype(jnp.int32),
         negsamples.astype(jnp.int32)], axis=0
    )
    idx_t = all_idx.reshape(C + 1, ns, 1, spt).transpose(1, 0, 2, 3)

    parts = _sc_dots_kernel(nc, ns, V, D, C, spt, first_embeddings.T, idx_t)
    out = _tc_loss_kernel(parts.reshape(nc, C * B // 1024, 1024), B)
    return out[0, 0]


# speedup vs baseline: 1.0100x; 1.0068x over previous
import functools

import jax
import jax.numpy as jnp
from jax import lax
from jax.experimental import pallas as pl
from jax.experimental.pallas import tpu as pltpu
from jax.experimental.pallas import tpu_sc as plsc


def _sc_dots_kernel(nc, ns, V, D, C, spt, table_t, idx_t):
    dpc = D // nc
    mesh = plsc.VectorSubcoreMesh(core_axis_name="c", subcore_axis_name="s")

    @functools.partial(
        pl.kernel,
        mesh=mesh,
        compiler_params=pltpu.CompilerParams(
            use_tc_tiling_on_sc=True, needs_layout_passes=False
        ),
        out_type=jax.ShapeDtypeStruct((nc, ns, C, spt), jnp.float32),
        scratch_types=[
            pltpu.VMEM_SHARED((V,), jnp.float32),
            pltpu.VMEM((1, (C + 1) * spt), jnp.int32),
            pltpu.VMEM((1, (C + 1) * spt), jnp.float32),
            pltpu.VMEM((C, 1, spt), jnp.float32),
            pltpu.SemaphoreType.DMA,
            pltpu.SemaphoreType.DMA,
            pltpu.SemaphoreType.DMA,
        ],
    )
    def k(tab, idx_h, out_h, sp0, idx_v, val_v, acc_v, semA, semB, semG):
        cid = lax.axis_index("c")
        sid = lax.axis_index("s")
        d0 = cid * dpc
        pltpu.sync_copy(idx_h.at[sid], idx_v)

        zeros16 = jnp.zeros((16,), jnp.float32)

        def zblk(b, _):
            for c in range(C):
                acc_v[c, 0, pl.ds(b * 16, 16)] = zeros16
            return 0

        lax.fori_loop(0, spt // 16, zblk, 0)

        def stage(d):
            pltpu.async_copy(tab.at[d], sp0, semA)

        def drain_row():
            pltpu.make_async_copy(tab.at[0], sp0, semA).wait()

        @pl.when(sid == 0)
        def _():
            stage(d0)

        def d_body(dl, _):
            @pl.when(sid == 0)
            def _():
                drain_row()

            plsc.subcore_barrier()

            pltpu.async_copy(sp0.at[idx_v.at[0]], val_v.at[0], semG).wait()

            plsc.subcore_barrier()

            @pl.when((sid == 0) & (dl + 1 < dpc))
            def _():
                stage(d0 + dl + 1)

            def blk(b, _):
                s0 = b * 16
                v0 = val_v[0, pl.ds(s0, 16)]
                for c in range(C):
                    acc_v[c, 0, pl.ds(s0, 16)] = (
                        acc_v[c, 0, pl.ds(s0, 16)]
                        + v0 * val_v[0, pl.ds((c + 1) * spt + s0, 16)]
                    )
                return 0

            lax.fori_loop(0, spt // 16, blk, 0)
            return 0

        lax.fori_loop(0, dpc, d_body, 0)

        for c in range(C):
            pltpu.sync_copy(acc_v.at[c, 0], out_h.at[cid, sid, c])

    return k(table_t, idx_t)


def _tc_loss_kernel(parts, batch):
    def body(x_ref, o_ref):
        x = x_ref[0] + x_ref[1]
        ls = jnp.minimum(x, 0.0) - jnp.log1p(jnp.exp(-jnp.abs(x)))
        o_ref[0, 0] = -jnp.sum(ls) / batch

    return pl.pallas_call(
        body,
        out_shape=jax.ShapeDtypeStruct((1, 1), jnp.float32),
        out_specs=pl.BlockSpec(memory_space=pltpu.SMEM),
    )(parts)


def kernel(v_i, v_j, negsamples, device, first_embeddings):
    B = v_i.shape[0]
    V, D = first_embeddings.shape
    C = negsamples.shape[0] + 1

    info = plsc.get_sparse_core_info()
    nc, ns = info.num_cores, info.num_subcores
    spt = B // ns

    all_idx = jnp.concatenate(
        [v_i[None].astype(jnp.int32), v_j[None].astype(jnp.int32),
         negsamples.astype(jnp.int32)], axis=0
    )
    idx_t = (all_idx.reshape(C + 1, ns, spt).transpose(1, 0, 2)
             .reshape(ns, 1, (C + 1) * spt))

    parts = _sc_dots_kernel(nc, ns, V, D, C, spt, first_embeddings.T, idx_t)
    out = _tc_loss_kernel(parts.reshape(nc, C * B // 1024, 1024), B)
    return out[0, 0]
